# baseline (speedup 1.0000x reference)
import functools

import jax
import jax.numpy as jnp
from jax import lax
from jax.experimental import pallas as pl
from jax.experimental.pallas import tpu as pltpu
from jax.experimental.pallas import tpu_sc as plsc

VOCAB = 1_000_000
EMB = 100


ROWBLK = 16384


def _rowmax_body(t_ref, o_ref):
    o_ref[...] = jnp.max(t_ref[...], axis=1)


def _rowmax(table):
    nvocab = table.shape[0]
    return pl.pallas_call(
        _rowmax_body,
        grid=(pl.cdiv(nvocab, ROWBLK),),
        in_specs=[pl.BlockSpec((ROWBLK, EMB), lambda i: (i, 0))],
        out_specs=pl.BlockSpec((ROWBLK,), lambda i: (i,)),
        out_shape=jax.ShapeDtypeStruct((nvocab,), jnp.float32),
    )(table)


NC, NS = 2, 16
NW = NC * NS
IDXW = 128
RSTEP = 16


def _sc_gather(rowmax, text):
    nrows, seqlen = text.shape
    rows_per_w = nrows // NW
    steps = rows_per_w // RSTEP

    @functools.partial(
        pl.kernel,
        out_type=jax.ShapeDtypeStruct((nrows, seqlen), jnp.float32),
        mesh=plsc.VectorSubcoreMesh(core_axis_name="c", subcore_axis_name="s"),
        scratch_types=[
            pltpu.VMEM((2 * RSTEP, seqlen), jnp.int32),
            pltpu.VMEM((2 * RSTEP, seqlen), jnp.float32),
            pltpu.SemaphoreType.DMA((2,)),
            pltpu.SemaphoreType.DMA((2,)),
            pltpu.SemaphoreType.DMA((2,)),
        ],
    )
    def k(rowmax_hbm, text_hbm, out_hbm, idx_v, val_v, isem, gsem, osem):
        w = lax.axis_index("s") * NC + lax.axis_index("c")
        base = w * rows_per_w

        def idx_copy(g, slot):
            return pltpu.make_async_copy(
                text_hbm.at[pl.ds(base + g * RSTEP, RSTEP)],
                idx_v.at[pl.ds(slot * RSTEP, RSTEP)],
                isem.at[slot],
            )

        def out_copy(g, slot):
            return pltpu.make_async_copy(
                val_v.at[pl.ds(slot * RSTEP, RSTEP)],
                out_hbm.at[pl.ds(base + g * RSTEP, RSTEP)],
                osem.at[slot],
            )

        idx_copy(0, 0).start()

        def body(g, carry):
            cur = lax.rem(g, 2)
            nxt = 1 - cur

            @pl.when(g + 1 < steps)
            def _():
                idx_copy(g + 1, nxt).start()

            idx_copy(g, cur).wait()

            @pl.when(g >= 2)
            def _():
                out_copy(g - 2, cur).wait()

            def gather_block(slot):
                cps = []
                for r in range(RSTEP):
                    cps.append(pltpu.async_copy(
                        rowmax_hbm.at[idx_v.at[slot * RSTEP + r]],
                        val_v.at[slot * RSTEP + r],
                        gsem.at[slot],
                    ))
                for cp in cps:
                    cp.wait()

            @pl.when(cur == 0)
            def _():
                gather_block(0)

            @pl.when(cur == 1)
            def _():
                gather_block(1)

            out_copy(g, cur).start()
            return carry

        lax.fori_loop(0, steps, body, 0, unroll=2)

        for gg in (steps - 2, steps - 1):
            out_copy(gg, gg % 2).wait()

    return k(rowmax, text)


BBLK = 1024


def _head_body(s_ref, wt_ref, b_ref, o_ref):
    s = s_ref[...]
    shifted = jnp.concatenate([s[:, 1:], s[:, :1]], axis=1)
    act = jnp.maximum(jnp.maximum(s, shifted), 0.0)
    o_ref[...] = (
        jnp.dot(act, wt_ref[...], preferred_element_type=jnp.float32) + b_ref[...]
    )


def _head(s, wt, b2):
    bsz, seqlen = s.shape
    out_dim = wt.shape[1]
    return pl.pallas_call(
        _head_body,
        grid=(bsz // BBLK,),
        in_specs=[
            pl.BlockSpec((BBLK, seqlen), lambda i: (i, 0)),
            pl.BlockSpec((seqlen, out_dim), lambda i: (0, 0)),
            pl.BlockSpec((1, out_dim), lambda i: (0, 0)),
        ],
        out_specs=pl.BlockSpec((BBLK, out_dim), lambda i: (i, 0)),
        out_shape=jax.ShapeDtypeStruct((bsz, out_dim), jnp.float32),
    )(s, wt, b2)


def kernel(text, table, W, b):
    bsz, seqlen = text.shape
    rm = _rowmax(table)
    textr = text.reshape(-1, IDXW).astype(jnp.int32)
    sflat = _sc_gather(rm, textr)
    s = sflat.reshape(bsz, seqlen)
    wt = jnp.pad(W.T, ((0, 1), (0, 0)))
    out = _head(s, wt, b.reshape(1, -1))
    return out


# speedup vs baseline: 1.0199x; 1.0199x over previous
import functools

import jax
import jax.numpy as jnp
from jax import lax
from jax.experimental import pallas as pl
from jax.experimental.pallas import tpu as pltpu
from jax.experimental.pallas import tpu_sc as plsc

VOCAB = 1_000_000
EMB = 100


ROWBLK = 16384


def _rowmax_body(t_ref, o_ref):
    o_ref[...] = jnp.max(t_ref[...], axis=1)


def _rowmax(table):
    nvocab = table.shape[0]
    return pl.pallas_call(
        _rowmax_body,
        grid=(pl.cdiv(nvocab, ROWBLK),),
        in_specs=[pl.BlockSpec((ROWBLK, EMB), lambda i: (i, 0))],
        out_specs=pl.BlockSpec((ROWBLK,), lambda i: (i,)),
        out_shape=jax.ShapeDtypeStruct((nvocab,), jnp.float32),
    )(table)


NC, NS = 2, 16
NW = NC * NS
IDXW = 128
RSTEP = 16


def _sc_gather(rowmax, text):
    nrows, seqlen = text.shape
    rows_per_w = nrows // NW
    steps = rows_per_w // RSTEP

    @functools.partial(
        pl.kernel,
        out_type=jax.ShapeDtypeStruct((nrows, seqlen), jnp.float32),
        mesh=plsc.VectorSubcoreMesh(core_axis_name="c", subcore_axis_name="s"),
        scratch_types=[
            pltpu.VMEM((2 * RSTEP, seqlen), jnp.int32),
            pltpu.VMEM((2 * RSTEP, seqlen), jnp.float32),
            pltpu.SemaphoreType.DMA((2,)),
            pltpu.SemaphoreType.DMA((2,)),
            pltpu.SemaphoreType.DMA((2,)),
        ],
    )
    def k(rowmax_hbm, text_hbm, out_hbm, idx_v, val_v, isem, gsem, osem):
        w = lax.axis_index("s") * NC + lax.axis_index("c")
        base = w * rows_per_w

        def idx_copy(g, slot):
            return pltpu.make_async_copy(
                text_hbm.at[pl.ds(base + g * RSTEP, RSTEP)],
                idx_v.at[pl.ds(slot * RSTEP, RSTEP)],
                isem.at[slot],
            )

        def out_copy(g, slot):
            return pltpu.make_async_copy(
                val_v.at[pl.ds(slot * RSTEP, RSTEP)],
                out_hbm.at[pl.ds(base + g * RSTEP, RSTEP)],
                osem.at[slot],
            )

        def fire(slot):
            for r in range(RSTEP):
                pltpu.async_copy(
                    rowmax_hbm.at[idx_v.at[slot * RSTEP + r]],
                    val_v.at[slot * RSTEP + r],
                    gsem.at[slot],
                )

        def drain(slot):
            for r in range(RSTEP):
                pltpu.make_async_copy(
                    rowmax_hbm.at[idx_v.at[slot * RSTEP + r]],
                    val_v.at[slot * RSTEP + r],
                    gsem.at[slot],
                ).wait()

        idx_copy(0, 0).start()

        def body(g, carry):
            cur = lax.rem(g, 2)
            prv = 1 - cur

            idx_copy(g, cur).wait()

            @pl.when(g >= 2)
            def _():
                out_copy(g - 2, cur).wait()

            @pl.when(cur == 0)
            def _():
                fire(0)

            @pl.when(cur == 1)
            def _():
                fire(1)

            @pl.when(jnp.logical_and(g >= 1, cur == 1))
            def _():
                drain(0)

            @pl.when(jnp.logical_and(g >= 1, cur == 0))
            def _():
                drain(1)

            @pl.when(g >= 1)
            def _():
                out_copy(g - 1, prv).start()

            @pl.when(g + 1 < steps)
            def _():
                idx_copy(g + 1, prv).start()

            return carry

        lax.fori_loop(0, steps, body, 0, unroll=2)

        lslot = (steps - 1) % 2
        drain(lslot)
        out_copy(steps - 1, lslot).start()
        out_copy(steps - 2, 1 - lslot).wait()
        out_copy(steps - 1, lslot).wait()

    return k(rowmax, text)


BBLK = 1024


def _head_body(s_ref, wt_ref, b_ref, o_ref):
    s = s_ref[...]
    shifted = jnp.concatenate([s[:, 1:], s[:, :1]], axis=1)
    act = jnp.maximum(jnp.maximum(s, shifted), 0.0)
    o_ref[...] = (
        jnp.dot(act, wt_ref[...], preferred_element_type=jnp.float32) + b_ref[...]
    )


def _head(s, wt, b2):
    bsz, seqlen = s.shape
    out_dim = wt.shape[1]
    return pl.pallas_call(
        _head_body,
        grid=(bsz // BBLK,),
        in_specs=[
            pl.BlockSpec((BBLK, seqlen), lambda i: (i, 0)),
            pl.BlockSpec((seqlen, out_dim), lambda i: (0, 0)),
            pl.BlockSpec((1, out_dim), lambda i: (0, 0)),
        ],
        out_specs=pl.BlockSpec((BBLK, out_dim), lambda i: (i, 0)),
        out_shape=jax.ShapeDtypeStruct((bsz, out_dim), jnp.float32),
    )(s, wt, b2)


def kernel(text, table, W, b):
    bsz, seqlen = text.shape
    rm = _rowmax(table)
    textr = text.reshape(-1, IDXW).astype(jnp.int32)
    sflat = _sc_gather(rm, textr)
    s = sflat.reshape(bsz, seqlen)
    wt = jnp.pad(W.T, ((0, 1), (0, 0)))
    out = _head(s, wt, b.reshape(1, -1))
    return out
